# baseline (speedup 1.0000x reference)
import dataclasses
import functools

import jax
import jax.numpy as jnp
from jax import lax
from jax.experimental import pallas as pl
from jax.experimental.pallas import tpu as pltpu
from jax.experimental.pallas import tpu_sc as plsc

_NC = 2
_NS = 16
_NW = _NC * _NS
_LANES = 16
_CHUNK = 64
_NBUF = 4
_PREF = 2
_STAGE = 64
_EPS = 1e-12

_GDN = lax.GatherDimensionNumbers(
    offset_dims=(), collapsed_slice_dims=(0,), start_index_map=(0,))


def _splat_lane(vec, j):
    idx = jnp.full((_LANES, 1), j, dtype=jnp.int32)
    return lax.gather(vec, idx, _GDN, (1,),
                      mode=lax.GatherScatterMode.PROMISE_IN_BOUNDS)


def _sc_embed_layernorm(word_table, pos_table, type_table, wids, pids, tids,
                        ln_weight, ln_bias):
    n = wids.shape[0]
    vocab, hidden = word_table.shape
    maxpos = pos_table.shape[0]
    types = type_table.shape[0]
    nh = hidden // _LANES
    nw = hidden // (2 * _LANES)
    per_w = n // _NW
    n_chunks = per_w // _CHUNK
    n_stage = maxpos // _STAGE
    mesh = plsc.VectorSubcoreMesh(core_axis_name="c", subcore_axis_name="s")
    cp = pltpu.CompilerParams()
    if "needs_layout_passes" in pltpu.CompilerParams.__dataclass_fields__:
        cp = dataclasses.replace(cp, needs_layout_passes=False)

    @functools.partial(
        pl.kernel,
        out_type=jax.ShapeDtypeStruct((n, hidden), jnp.float32),
        mesh=mesh,
        compiler_params=cp,
        scratch_types=(
            [
                pltpu.VMEM((per_w,), jnp.int32),
                pltpu.VMEM((per_w,), jnp.int32),
                pltpu.VMEM((per_w,), jnp.int32),
                pltpu.VMEM((types * maxpos * (hidden // 2),), jnp.int32),
                pltpu.VMEM((types, hidden), jnp.float32),
                pltpu.VMEM((hidden,), jnp.float32),
                pltpu.VMEM((hidden,), jnp.float32),
            ]
            + [pltpu.VMEM((_CHUNK, hidden), jnp.float32) for _ in range(_NBUF)]
            + [pltpu.SemaphoreType.DMA for _ in range(2 * _NBUF)]
        ),
    )
    def k(word_hbm, pos_hbm, type_hbm, wid_hbm, pid_hbm, tid_hbm, lnw_hbm,
          lnb_hbm, out_hbm, wid_v, cid_v, tid_v, comb_v, type_v,
          lnw_v, lnb_v, *bufs_and_sems):
        ws = bufs_and_sems[:_NBUF]
        sws = bufs_and_sems[_NBUF:2 * _NBUF]
        sos = bufs_and_sems[2 * _NBUF:]
        w = lax.axis_index("s") * _NC + lax.axis_index("c")
        base = w * per_w

        pltpu.sync_copy(wid_hbm.at[pl.ds(base, per_w)], wid_v)
        pltpu.sync_copy(pid_hbm.at[pl.ds(base, per_w)], cid_v)
        pltpu.sync_copy(tid_hbm.at[pl.ds(base, per_w)], tid_v)
        pltpu.sync_copy(lnw_hbm, lnw_v)
        pltpu.sync_copy(lnb_hbm, lnb_v)
        pltpu.sync_copy(type_hbm, type_v)

        wpr = hidden // 2
        @pl.loop(0, per_w, step=_LANES)
        def _(i):
            s = pl.ds(i, _LANES)
            cid_v[s] = (tid_v[s] * maxpos + cid_v[s]) * wpr

        stage_v = ws[0]
        for s in range(n_stage):
            pltpu.sync_copy(pos_hbm.at[pl.ds(s * _STAGE, _STAGE)], stage_v)

            @pl.loop(0, _STAGE)
            def _(r):
                prow = [stage_v[r, pl.ds(h * _LANES, _LANES)]
                        for h in range(nh)]
                for t in range(types):
                    trow = [type_v[t, pl.ds(h * _LANES, _LANES)]
                            for h in range(nh)]
                    for g in range(nw):
                        a = prow[2 * g] + trow[2 * g]
                        b = prow[2 * g + 1] + trow[2 * g + 1]
                        packed = plsc.pack(
                            a, b, format=plsc.PackFormat.INTERLEAVED)
                        comb_v[pl.ds((t * maxpos + s * _STAGE + r) * wpr
                                     + g * _LANES, _LANES)] = (
                            plsc.bitcast(packed, jnp.int32))

        nh_r = [lnw_v[pl.ds(h * _LANES, _LANES)] for h in range(nh)]
        nb_r = [lnb_v[pl.ds(h * _LANES, _LANES)] for h in range(nh)]
        col_r = [lax.iota(jnp.int32, _LANES) + (g * _LANES) for g in range(nw)]
        inv_h = jnp.float32(1.0 / hidden)

        def gather_copy(chunk, b):
            off = pl.multiple_of(chunk * _CHUNK, _CHUNK)
            return pltpu.make_async_copy(
                word_hbm.at[wid_v.at[pl.ds(off, _CHUNK)]], ws[b], sws[b])

        def out_copy(chunk, b):
            off = pl.multiple_of(chunk * _CHUNK, _CHUNK)
            return pltpu.make_async_copy(
                ws[b], out_hbm.at[pl.ds(base + off, _CHUNK)], sos[b])

        def ln_token(b, t, cidg, j):
            row = _splat_lane(cidg, j)
            e = []
            for g in range(nw):
                words = plsc.load_gather(comb_v, [row + col_r[g]])
                a, bb = plsc.unpack(plsc.bitcast(words, jnp.bfloat16),
                                    format=plsc.PackFormat.INTERLEAVED)
                slc_a = (t, pl.ds((2 * g) * _LANES, _LANES))
                slc_b = (t, pl.ds((2 * g + 1) * _LANES, _LANES))
                e.append(ws[b][*slc_a] + a.astype(jnp.float32))
                e.append(ws[b][*slc_b] + bb.astype(jnp.float32))
            acc_s = e[0]
            acc_q = e[0] * e[0]
            for h in range(1, nh):
                acc_s = acc_s + e[h]
                acc_q = acc_q + e[h] * e[h]
            mean = jnp.sum(acc_s) * inv_h
            var = jnp.sum(acc_q) * inv_h - mean * mean
            x = jnp.full((_LANES,), var + _EPS, dtype=jnp.float32)
            i = lax.bitcast_convert_type(x, jnp.int32)
            i = jnp.int32(0x5F3759DF) - lax.shift_right_logical(i, 1)
            y = lax.bitcast_convert_type(i, jnp.float32)
            for _ in range(3):
                y = y * (1.5 - 0.5 * x * y * y)
            m = jnp.full((_LANES,), mean, dtype=jnp.float32)
            for h in range(nh):
                ws[b][t, pl.ds(h * _LANES, _LANES)] = (
                    (e[h] - m) * (y * nh_r[h]) + nb_r[h])

        for b in range(_PREF):
            gather_copy(b, b).start()

        @pl.loop(0, n_chunks, step=_NBUF)
        def _(g):
            for b in range(_NBUF):
                kk = g + b

                @pl.when(kk + _PREF < n_chunks)
                def _():
                    nb = (b + _PREF) % _NBUF

                    @pl.when(kk >= _NBUF - _PREF)
                    def _():
                        out_copy(kk - (_NBUF - _PREF), nb).wait()
                    gather_copy(kk + _PREF, nb).start()

                gather_copy(kk, b).wait()

                @pl.loop(0, _CHUNK, step=_LANES)
                def _(t0):
                    cidg = cid_v[pl.ds(
                        pl.multiple_of(kk * _CHUNK, _CHUNK) + t0, _LANES)]
                    for j in range(_LANES):
                        ln_token(b, t0 + j, cidg, j)

                out_copy(kk, b).start()

        for b in range(_NBUF):
            out_copy(n_chunks - _NBUF + b, (n_chunks - _NBUF + b) % _NBUF).wait()

    return k(word_table, pos_table, type_table, wids, pids, tids,
             ln_weight, ln_bias)


def kernel(input_ids, position_ids, token_type_ids, word_table, pos_table,
           type_table, ln_weight, ln_bias):
    b, l = input_ids.shape
    hidden = word_table.shape[1]
    n = b * l
    wids = input_ids.reshape(n).astype(jnp.int32)
    pids = position_ids.reshape(n).astype(jnp.int32)
    tids = token_type_ids.reshape(n).astype(jnp.int32)

    out = _sc_embed_layernorm(word_table, pos_table, type_table, wids, pids,
                              tids, ln_weight, ln_bias)
    return out.reshape(b, l, hidden)


# speedup vs baseline: 2.2706x; 2.2706x over previous
import dataclasses
import functools

import jax
import jax.numpy as jnp
from jax import lax
from jax.experimental import pallas as pl
from jax.experimental.pallas import tpu as pltpu
from jax.experimental.pallas import tpu_sc as plsc

_NC = 2
_NS = 16
_NW = _NC * _NS
_LANES = 16
_CHUNK = 128
_EPS = 1e-12


def _build_comb(pos_table, type_table):
    maxpos, hidden = pos_table.shape
    types = type_table.shape[0]

    def body(pos_ref, type_ref, o_ref):
        for t in range(types):
            o_ref[t * maxpos:(t + 1) * maxpos, :] = (
                pos_ref[...] + type_ref[t:t + 1, :]
            )

    return pl.pallas_call(
        body,
        out_shape=jax.ShapeDtypeStruct((types * maxpos, hidden), jnp.float32),
    )(pos_table, type_table)


def _sc_embed_layernorm(word_table, comb, wids, pids, tids, ln_weight,
                        ln_bias, type_count):
    n = wids.shape[0]
    hidden = word_table.shape[1]
    maxpos = comb.shape[0] // type_count
    per_w = n // _NW
    n_chunks = per_w // _CHUNK
    mesh = plsc.VectorSubcoreMesh(core_axis_name="c", subcore_axis_name="s")
    cp = pltpu.CompilerParams()
    if "needs_layout_passes" in pltpu.CompilerParams.__dataclass_fields__:
        cp = dataclasses.replace(cp, needs_layout_passes=False)

    @functools.partial(
        pl.kernel,
        out_type=jax.ShapeDtypeStruct((n, hidden), jnp.float32),
        mesh=mesh,
        compiler_params=cp,
        scratch_types=[
            pltpu.VMEM((per_w,), jnp.int32),
            pltpu.VMEM((per_w,), jnp.int32),
            pltpu.VMEM((per_w,), jnp.int32),
            pltpu.VMEM((_CHUNK, hidden), jnp.float32),
            pltpu.VMEM((_CHUNK, hidden), jnp.float32),
            pltpu.VMEM((_CHUNK, hidden), jnp.float32),
            pltpu.VMEM((_CHUNK, hidden), jnp.float32),
            pltpu.VMEM((_CHUNK, hidden), jnp.float32),
            pltpu.VMEM((_CHUNK, hidden), jnp.float32),
            pltpu.VMEM((hidden,), jnp.float32),
            pltpu.VMEM((hidden,), jnp.float32),
            pltpu.SemaphoreType.DMA,
            pltpu.SemaphoreType.DMA,
            pltpu.SemaphoreType.DMA,
            pltpu.SemaphoreType.DMA,
            pltpu.SemaphoreType.DMA,
            pltpu.SemaphoreType.DMA,
        ],
    )
    def k(word_hbm, comb_hbm, wid_hbm, pid_hbm, tid_hbm, lnw_hbm, lnb_hbm,
          out_hbm, wid_v, cid_v, tid_v, w0, w1, c0, c1, o0, o1, lnw_v, lnb_v,
          sw0, sw1, sc0, sc1, so0, so1):
        w = lax.axis_index("s") * _NC + lax.axis_index("c")
        base = w * per_w
        ws, cs, os_ = (w0, w1), (c0, c1), (o0, o1)
        sws, scs, sos = (sw0, sw1), (sc0, sc1), (so0, so1)

        pltpu.sync_copy(wid_hbm.at[pl.ds(base, per_w)], wid_v)
        pltpu.sync_copy(pid_hbm.at[pl.ds(base, per_w)], cid_v)
        pltpu.sync_copy(tid_hbm.at[pl.ds(base, per_w)], tid_v)
        pltpu.sync_copy(lnw_hbm, lnw_v)
        pltpu.sync_copy(lnb_hbm, lnb_v)

        @pl.loop(0, per_w, step=_LANES)
        def _(i):
            s = pl.ds(i, _LANES)
            cid_v[s] = cid_v[s] + tid_v[s] * maxpos

        def gather_copies(chunk, b):
            off = pl.multiple_of(chunk * _CHUNK, _CHUNK)
            return (
                pltpu.make_async_copy(
                    word_hbm.at[wid_v.at[pl.ds(off, _CHUNK)]], ws[b], sws[b]),
                pltpu.make_async_copy(
                    comb_hbm.at[cid_v.at[pl.ds(off, _CHUNK)]], cs[b], scs[b]),
            )

        def out_copy(chunk, b):
            off = pl.multiple_of(chunk * _CHUNK, _CHUNK)
            return pltpu.make_async_copy(
                os_[b], out_hbm.at[pl.ds(base + off, _CHUNK)], sos[b])

        nh = hidden // _LANES
        lnw_r = [lnw_v[pl.ds(h * _LANES, _LANES)] for h in range(nh)]
        lnb_r = [lnb_v[pl.ds(h * _LANES, _LANES)] for h in range(nh)]
        inv_h = jnp.float32(1.0 / hidden)

        def ln_token(b, t):
            e = []
            for h in range(nh):
                slc = (t, pl.ds(h * _LANES, _LANES))
                e.append(ws[b][*slc] + cs[b][*slc])
            acc_s = e[0]
            acc_q = e[0] * e[0]
            for h in range(1, nh):
                acc_s = acc_s + e[h]
                acc_q = acc_q + e[h] * e[h]
            mean = jnp.sum(acc_s) * inv_h
            var = jnp.sum(acc_q) * inv_h - mean * mean
            x = jnp.full((_LANES,), var + _EPS, dtype=jnp.float32)
            i = lax.bitcast_convert_type(x, jnp.int32)
            i = jnp.int32(0x5F3759DF) - lax.shift_right_logical(i, 1)
            y = lax.bitcast_convert_type(i, jnp.float32)
            for _ in range(3):
                y = y * (1.5 - 0.5 * x * y * y)
            m = jnp.full((_LANES,), mean, dtype=jnp.float32)
            for h in range(nh):
                slc = (t, pl.ds(h * _LANES, _LANES))
                os_[b][*slc] = (e[h] - m) * (y * lnw_r[h]) + lnb_r[h]

        for cp_ in gather_copies(0, 0):
            cp_.start()

        @pl.loop(0, n_chunks, step=2)
        def _(g):
            for b in range(2):
                kk = g + b

                @pl.when(kk + 1 < n_chunks)
                def _():
                    for cp_ in gather_copies(kk + 1, 1 - b):
                        cp_.start()

                for cp_ in gather_copies(kk, b):
                    cp_.wait()

                @pl.when(kk >= 2)
                def _():
                    out_copy(kk - 2, b).wait()

                @pl.loop(0, _CHUNK, step=2)
                def _(t):
                    ln_token(b, t)
                    ln_token(b, t + 1)

                out_copy(kk, b).start()

        out_copy(n_chunks - 2, 0).wait()
        out_copy(n_chunks - 1, 1).wait()

    return k(word_table, comb, wids, pids, tids, ln_weight, ln_bias)


def kernel(input_ids, position_ids, token_type_ids, word_table, pos_table,
           type_table, ln_weight, ln_bias):
    b, l = input_ids.shape
    hidden = word_table.shape[1]
    n = b * l
    wids = input_ids.reshape(n).astype(jnp.int32)
    pids = position_ids.reshape(n).astype(jnp.int32)
    tids = token_type_ids.reshape(n).astype(jnp.int32)

    comb = _build_comb(pos_table, type_table)
    out = _sc_embed_layernorm(word_table, comb, wids, pids, tids,
                              ln_weight, ln_bias, type_table.shape[0])
    return out.reshape(b, l, hidden)
